# baseline (speedup 1.0000x reference)
import functools

import jax
import jax.numpy as jnp
from jax import lax
from jax.experimental import pallas as pl
from jax.experimental.pallas import tpu as pltpu
from jax.experimental.pallas import tpu_sc as plsc

NC = 2
NS = 16
NW = NC * NS

_W0 = 0.3535533905932738
_W1 = 0.6123724356957945

_CHUNK = 128
_HALF = _CHUNK // 2


@functools.lru_cache(maxsize=None)
def _make_segsum(n, tch, d):
    assert tch % NW == 0
    cpt = tch // NW
    assert cpt % 2 == 0
    np_ = ((n + NS * 8 - 1) // (NS * 8)) * (NS * 8)
    rows_pt = np_ // NS
    assert d % 16 == 0
    ngr = _CHUNK // 16

    mesh = plsc.VectorSubcoreMesh(core_axis_name="c", subcore_axis_name="s")

    @functools.partial(
        pl.kernel,
        out_type=[
            jax.ShapeDtypeStruct((np_, d), jnp.float32),
            jax.ShapeDtypeStruct((np_, d), jnp.float32),
        ],
        mesh=mesh,
        scratch_types=[
            pltpu.VMEM_SHARED((np_, d), jnp.float32),
            pltpu.VMEM((cpt, _CHUNK), jnp.int32),
            pltpu.VMEM((cpt, _CHUNK), jnp.int32),
            pltpu.VMEM((cpt, _CHUNK), jnp.float32),
            pltpu.VMEM((2, _CHUNK, d), jnp.float32),
            pltpu.SemaphoreType.DMA((2,)),
            pltpu.SemaphoreType.DMA((2,)),
        ],
    )
    def segsum(x_hbm, src_hbm, dst_hbm, e_hbm, z_hbm,
               out0, out1, acc, src_v, dst_v, w_v, rows_v, gsems, ssems):
        c = lax.axis_index("c")
        s = lax.axis_index("s")
        wid = s * NC + c
        r0 = s * rows_pt

        def gather(ci, b):
            pltpu.async_copy(x_hbm.at[src_v.at[ci]], rows_v.at[b], gsems.at[b])

        def scatter(ci, b):
            pltpu.async_copy(rows_v.at[b], acc.at[dst_v.at[ci]],
                             ssems.at[b], add=True)

        def drain(semref, b):
            pltpu.make_async_copy(x_hbm.at[pl.ds(0, _CHUNK)],
                                  rows_v.at[b], semref.at[b]).wait()

        pltpu.sync_copy(z_hbm.at[pl.ds(r0, rows_pt)], acc.at[pl.ds(r0, rows_pt)])
        pltpu.sync_copy(src_hbm.at[wid], src_v)
        pltpu.sync_copy(dst_hbm.at[wid], dst_v)
        pltpu.sync_copy(e_hbm.at[wid], w_v)
        plsc.subcore_barrier()

        def compute(ci, b):
            def group_body(g, _):
                sl = pl.ds(g * 16, 16)
                wv = jnp.where(src_v[ci, sl] == dst_v[ci, sl],
                               jnp.float32(-1.0), w_v[ci, sl])
                for j in range(16):
                    wsp = jnp.full((16,), wv[j], jnp.float32)
                    row = g * 16 + j
                    for q in range(d // 16):
                        csl = pl.ds(q * 16, 16)
                        rows_v[b, row, csl] = rows_v[b, row, csl] * wsp
                return 0

            lax.fori_loop(0, ngr, group_body, 0)

        def piter(ci, _):
            b = ci % 2

            @pl.when(ci < cpt)
            def _():
                @pl.when(ci >= 2)
                def _():
                    drain(ssems, b)
                gather(ci, b)

            @pl.when(ci >= 1)
            def _():
                cj = ci - 1
                bj = (ci - 1) % 2
                drain(gsems, bj)
                compute(cj, bj)
                scatter(cj, bj)

            return 0

        lax.fori_loop(0, cpt + 1, piter, 0)
        drain(ssems, 0)
        drain(ssems, 1)
        plsc.subcore_barrier()

        @pl.when(c == 0)
        def _():
            pltpu.sync_copy(acc.at[pl.ds(r0, rows_pt)], out0.at[pl.ds(r0, rows_pt)])

        @pl.when(c == 1)
        def _():
            pltpu.sync_copy(acc.at[pl.ds(r0, rows_pt)], out1.at[pl.ds(r0, rows_pt)])

    return segsum


def _row_grid(n, d, n_in, body, out_shape=None, block_rows=2000,
              out_dtype=jnp.float32):
    grid = (pl.cdiv(n, block_rows),)
    in_specs = [pl.BlockSpec((block_rows, d), lambda i: (i, 0))] * n_in
    if out_shape is None:
        out_shape = (n, d)
    out_spec = pl.BlockSpec((block_rows, out_shape[1]), lambda i: (i, 0))
    return pl.pallas_call(
        body,
        grid=grid,
        in_specs=in_specs,
        out_specs=out_spec,
        out_shape=jax.ShapeDtypeStruct(out_shape, out_dtype),
    )


def _x1_body(x0, a00, a01, o):
    o[...] = x0[...] + (a00[...] + a01[...]) * jnp.float32(1.0 / 3.0)


def _x2_body(x0, a00, a01, a10, a11, o):
    o[...] = x0[...] + (a10[...] + a11[...]) - (a00[...] + a01[...]) * jnp.float32(1.0 / 3.0)


def _g_body(x0, x1, x2, a00, a01, a10, a11, a20, a21, o):
    x3 = (x0[...] + (a00[...] + a01[...]) - (a10[...] + a11[...])
          + (a20[...] + a21[...]))
    d = x0.shape[1]
    o[:, 0 * d:1 * d] = (x0[...] * jnp.float32(_W0)).astype(o.dtype)
    o[:, 1 * d:2 * d] = (x1[...] * jnp.float32(_W1)).astype(o.dtype)
    o[:, 2 * d:3 * d] = (x2[...] * jnp.float32(_W1)).astype(o.dtype)
    o[:, 3 * d:4 * d] = (x3 * jnp.float32(_W0)).astype(o.dtype)


def _mm_body(gi, gj, o):
    o[...] = lax.dot_general(
        gi[...], gj[...], (((1,), (1,)), ((), ())),
        preferred_element_type=jnp.float32,
    )


@functools.lru_cache(maxsize=None)
def _make_gram(n, k, bm, bn):
    grid = (pl.cdiv(n, bm), pl.cdiv(n, bn))
    return pl.pallas_call(
        _mm_body,
        grid=grid,
        in_specs=[
            pl.BlockSpec((bm, k), lambda i, j: (i, 0)),
            pl.BlockSpec((bn, k), lambda i, j: (j, 0)),
        ],
        out_specs=pl.BlockSpec((bm, bn), lambda i, j: (i, j)),
        out_shape=jax.ShapeDtypeStruct((n, n), jnp.float32),
        compiler_params=pltpu.CompilerParams(
            dimension_semantics=("parallel", "parallel"),
        ),
    )


def kernel(h, e, edge_index):
    n, d = h.shape
    ev = e.shape[0]
    src = edge_index[0]
    dst = edge_index[1]
    ew = e.reshape(ev)
    np_ = ((n + NS * 8 - 1) // (NS * 8)) * (NS * 8)
    zeros = jnp.zeros((np_, d), jnp.float32)

    tch = -(-ev // _CHUNK)
    tch = -(-tch // (NW * 2)) * (NW * 2)
    evp = tch * _CHUNK
    if evp > ev:
        padn = evp - ev
        pad_dst = jnp.arange(padn, dtype=jnp.int32) % n
        pad_src = (pad_dst + 1) % n
        src = jnp.concatenate([src, pad_src])
        dst = jnp.concatenate([dst, pad_dst])
        ew = jnp.concatenate([ew, jnp.zeros((padn,), jnp.float32)])
    cpt = tch // NW
    src2 = src.reshape(NW, cpt, _CHUNK)
    dst2 = dst.reshape(NW, cpt, _CHUNK)
    ew2 = ew.reshape(NW, cpt, _CHUNK)

    segsum = _make_segsum(n, tch, d)
    a00, a01 = segsum(h, src2, dst2, ew2, zeros)
    x1 = _row_grid(n, d, 3, _x1_body)(h, a00, a01)
    a10, a11 = segsum(x1, src2, dst2, ew2, zeros)
    x2 = _row_grid(n, d, 5, _x2_body)(h, a00, a01, a10, a11)
    a20, a21 = segsum(x2, src2, dst2, ew2, zeros)
    g = _row_grid(n, d, 9, _g_body, out_shape=(n, 4 * d),
                  out_dtype=jnp.bfloat16)(
        h, x1, x2, a00, a01, a10, a11, a20, a21)
    return _make_gram(n, 4 * d, 2048, 2048)(g, g)


# speedup vs baseline: 1.8142x; 1.8142x over previous
import functools

import jax
import jax.numpy as jnp
from jax import lax
from jax.experimental import pallas as pl
from jax.experimental.pallas import tpu as pltpu
from jax.experimental.pallas import tpu_sc as plsc

NC = 2
NS = 16
NW = NC * NS

_W0 = 0.3535533905932738
_W1 = 0.6123724356957945

_CHUNK = 128
_HALF = _CHUNK // 2


@functools.lru_cache(maxsize=None)
def _make_segsum(n, tch, d):
    assert tch % NW == 0
    cpt = tch // NW
    assert cpt % 2 == 0
    np_ = ((n + NS * 8 - 1) // (NS * 8)) * (NS * 8)
    rows_pt = np_ // NS
    assert d % 16 == 0
    ngr = _CHUNK // 16

    mesh = plsc.VectorSubcoreMesh(core_axis_name="c", subcore_axis_name="s")

    @functools.partial(
        pl.kernel,
        out_type=[
            jax.ShapeDtypeStruct((np_, d), jnp.float32),
            jax.ShapeDtypeStruct((np_, d), jnp.float32),
        ],
        mesh=mesh,
        scratch_types=[
            pltpu.VMEM_SHARED((np_, d), jnp.float32),
            pltpu.VMEM((cpt, _CHUNK), jnp.int32),
            pltpu.VMEM((cpt, _CHUNK), jnp.int32),
            pltpu.VMEM((cpt, _CHUNK), jnp.float32),
            pltpu.VMEM((2, _CHUNK, d), jnp.float32),
            pltpu.SemaphoreType.DMA((2,)),
            pltpu.SemaphoreType.DMA((2,)),
        ],
    )
    def segsum(x_hbm, src_hbm, dst_hbm, e_hbm, z_hbm,
               out0, out1, acc, src_v, dst_v, w_v, rows_v, gsems, ssems):
        c = lax.axis_index("c")
        s = lax.axis_index("s")
        wid = s * NC + c
        r0 = s * rows_pt

        def gather(ci, b):
            pltpu.async_copy(x_hbm.at[src_v.at[ci]], rows_v.at[b], gsems.at[b])

        def scatter(ci, b):
            pltpu.async_copy(rows_v.at[b], acc.at[dst_v.at[ci]],
                             ssems.at[b], add=True)

        def drain(semref, b):
            pltpu.make_async_copy(x_hbm.at[pl.ds(0, _CHUNK)],
                                  rows_v.at[b], semref.at[b]).wait()

        pltpu.sync_copy(z_hbm.at[pl.ds(r0, rows_pt)], acc.at[pl.ds(r0, rows_pt)])
        pltpu.sync_copy(src_hbm.at[wid], src_v)
        pltpu.sync_copy(dst_hbm.at[wid], dst_v)
        pltpu.sync_copy(e_hbm.at[wid], w_v)
        plsc.subcore_barrier()

        def compute(ci, b):
            def group_body(g, _):
                sl = pl.ds(g * 16, 16)
                wv = jnp.where(src_v[ci, sl] == dst_v[ci, sl],
                               jnp.float32(-1.0), w_v[ci, sl])
                for j in range(16):
                    wsp = jnp.full((16,), wv[j], jnp.float32)
                    row = g * 16 + j
                    for q in range(d // 16):
                        csl = pl.ds(q * 16, 16)
                        rows_v[b, row, csl] = rows_v[b, row, csl] * wsp
                return 0

            lax.fori_loop(0, ngr, group_body, 0)

        def step(ci, b):
            @pl.when(ci + 1 < cpt)
            def _():
                @pl.when(ci >= 1)
                def _():
                    drain(ssems, 1 - b)
                gather(ci + 1, 1 - b)
            drain(gsems, b)
            compute(ci, b)
            scatter(ci, b)

        gather(jnp.int32(0), 0)

        def outer(k, _):
            step(2 * k, 0)
            step(2 * k + 1, 1)
            return 0

        lax.fori_loop(0, cpt // 2, outer, 0)
        drain(ssems, 0)
        drain(ssems, 1)
        plsc.subcore_barrier()

        @pl.when(c == 0)
        def _():
            pltpu.sync_copy(acc.at[pl.ds(r0, rows_pt)], out0.at[pl.ds(r0, rows_pt)])

        @pl.when(c == 1)
        def _():
            pltpu.sync_copy(acc.at[pl.ds(r0, rows_pt)], out1.at[pl.ds(r0, rows_pt)])

    return segsum


def _row_grid(n, d, n_in, body, out_shape=None, block_rows=2000,
              out_dtype=jnp.float32):
    grid = (pl.cdiv(n, block_rows),)
    in_specs = [pl.BlockSpec((block_rows, d), lambda i: (i, 0))] * n_in
    if out_shape is None:
        out_shape = (n, d)
    out_spec = pl.BlockSpec((block_rows, out_shape[1]), lambda i: (i, 0))
    return pl.pallas_call(
        body,
        grid=grid,
        in_specs=in_specs,
        out_specs=out_spec,
        out_shape=jax.ShapeDtypeStruct(out_shape, out_dtype),
    )


def _x1_body(x0, a00, a01, o):
    o[...] = x0[...] + (a00[...] + a01[...]) * jnp.float32(1.0 / 3.0)


def _x2_body(x0, a00, a01, a10, a11, o):
    o[...] = x0[...] + (a10[...] + a11[...]) - (a00[...] + a01[...]) * jnp.float32(1.0 / 3.0)


def _g_body(x0, x1, x2, a00, a01, a10, a11, a20, a21, o):
    x3 = (x0[...] + (a00[...] + a01[...]) - (a10[...] + a11[...])
          + (a20[...] + a21[...]))
    d = x0.shape[1]
    o[:, 0 * d:1 * d] = (x0[...] * jnp.float32(_W0)).astype(o.dtype)
    o[:, 1 * d:2 * d] = (x1[...] * jnp.float32(_W1)).astype(o.dtype)
    o[:, 2 * d:3 * d] = (x2[...] * jnp.float32(_W1)).astype(o.dtype)
    o[:, 3 * d:4 * d] = (x3 * jnp.float32(_W0)).astype(o.dtype)


def _mm_body(gi, gj, o):
    o[...] = lax.dot_general(
        gi[...], gj[...], (((1,), (1,)), ((), ())),
        preferred_element_type=jnp.float32,
    )


@functools.lru_cache(maxsize=None)
def _make_gram(n, k, bm, bn):
    grid = (pl.cdiv(n, bm), pl.cdiv(n, bn))
    return pl.pallas_call(
        _mm_body,
        grid=grid,
        in_specs=[
            pl.BlockSpec((bm, k), lambda i, j: (i, 0)),
            pl.BlockSpec((bn, k), lambda i, j: (j, 0)),
        ],
        out_specs=pl.BlockSpec((bm, bn), lambda i, j: (i, j)),
        out_shape=jax.ShapeDtypeStruct((n, n), jnp.float32),
        compiler_params=pltpu.CompilerParams(
            dimension_semantics=("parallel", "parallel"),
        ),
    )


def kernel(h, e, edge_index):
    n, d = h.shape
    ev = e.shape[0]
    src = edge_index[0]
    dst = edge_index[1]
    ew = e.reshape(ev)
    np_ = ((n + NS * 8 - 1) // (NS * 8)) * (NS * 8)
    zeros = jnp.zeros((np_, d), jnp.float32)

    tch = -(-ev // _CHUNK)
    tch = -(-tch // (NW * 2)) * (NW * 2)
    evp = tch * _CHUNK
    if evp > ev:
        padn = evp - ev
        pad_dst = jnp.arange(padn, dtype=jnp.int32) % n
        pad_src = (pad_dst + 1) % n
        src = jnp.concatenate([src, pad_src])
        dst = jnp.concatenate([dst, pad_dst])
        ew = jnp.concatenate([ew, jnp.zeros((padn,), jnp.float32)])
    cpt = tch // NW
    src2 = src.reshape(NW, cpt, _CHUNK)
    dst2 = dst.reshape(NW, cpt, _CHUNK)
    ew2 = ew.reshape(NW, cpt, _CHUNK)

    segsum = _make_segsum(n, tch, d)
    a00, a01 = segsum(h, src2, dst2, ew2, zeros)
    x1 = _row_grid(n, d, 3, _x1_body)(h, a00, a01)
    a10, a11 = segsum(x1, src2, dst2, ew2, zeros)
    x2 = _row_grid(n, d, 5, _x2_body)(h, a00, a01, a10, a11)
    a20, a21 = segsum(x2, src2, dst2, ew2, zeros)
    g = _row_grid(n, d, 9, _g_body, out_shape=(n, 4 * d),
                  out_dtype=jnp.bfloat16)(
        h, x1, x2, a00, a01, a10, a11, a20, a21)
    return _make_gram(n, 4 * d, 2048, 2048)(g, g)


# speedup vs baseline: 2.0177x; 1.1122x over previous
import functools

import jax
import jax.numpy as jnp
from jax import lax
from jax.experimental import pallas as pl
from jax.experimental.pallas import tpu as pltpu
from jax.experimental.pallas import tpu_sc as plsc

NC = 2
NS = 16
NW = NC * NS

_W0 = 0.3535533905932738
_W1 = 0.6123724356957945

_CHUNK = 128
_HALF = _CHUNK // 2


@functools.lru_cache(maxsize=None)
def _make_segsum(n, tch, d):
    assert tch % NW == 0
    cpt = tch // NW
    assert cpt % 2 == 0
    np_ = ((n + NS * 8 - 1) // (NS * 8)) * (NS * 8)
    rows_pt = np_ // NS
    assert d % 16 == 0
    ngr = _CHUNK // 16

    mesh = plsc.VectorSubcoreMesh(core_axis_name="c", subcore_axis_name="s")

    @functools.partial(
        pl.kernel,
        out_type=[
            jax.ShapeDtypeStruct((np_, d), jnp.float32),
            jax.ShapeDtypeStruct((np_, d), jnp.float32),
        ],
        mesh=mesh,
        scratch_types=[
            pltpu.VMEM_SHARED((np_, d), jnp.float32),
            pltpu.VMEM((cpt, _CHUNK), jnp.int32),
            pltpu.VMEM((cpt, _CHUNK), jnp.int32),
            pltpu.VMEM((cpt, _CHUNK), jnp.float32),
            pltpu.VMEM((2, _CHUNK, d), jnp.float32),
            pltpu.SemaphoreType.DMA((2,)),
            pltpu.SemaphoreType.DMA((2,)),
        ],
    )
    def segsum(x_hbm, src_hbm, dst_hbm, e_hbm, z_hbm,
               out0, out1, acc, src_v, dst_v, w_v, rows_v, gsems, ssems):
        c = lax.axis_index("c")
        s = lax.axis_index("s")
        wid = s * NC + c
        r0 = s * rows_pt

        def gather(ci, b):
            pltpu.async_copy(x_hbm.at[src_v.at[ci]], rows_v.at[b], gsems.at[b])

        def scatter(ci, b):
            pltpu.async_copy(rows_v.at[b], acc.at[dst_v.at[ci]],
                             ssems.at[b], add=True)

        def drain(semref, b):
            pltpu.make_async_copy(x_hbm.at[pl.ds(0, _CHUNK)],
                                  rows_v.at[b], semref.at[b]).wait()

        pltpu.sync_copy(z_hbm.at[pl.ds(r0, rows_pt)], acc.at[pl.ds(r0, rows_pt)])
        pltpu.sync_copy(src_hbm.at[wid], src_v)
        pltpu.sync_copy(dst_hbm.at[wid], dst_v)
        pltpu.sync_copy(e_hbm.at[wid], w_v)

        def wbody(ci, _):
            for g in range(ngr):
                sl = pl.ds(g * 16, 16)
                w_v[ci, sl] = jnp.where(src_v[ci, sl] == dst_v[ci, sl],
                                        jnp.float32(-1.0), w_v[ci, sl])
            return 0

        lax.fori_loop(0, cpt, wbody, 0)
        plsc.subcore_barrier()

        def compute(ci, b):
            def group_body(g, _):
                sl = pl.ds(g * 16, 16)
                wv = w_v[ci, sl]
                for j in range(16):
                    wsp = jnp.full((16,), wv[j], jnp.float32)
                    row = g * 16 + j
                    for q in range(d // 16):
                        csl = pl.ds(q * 16, 16)
                        rows_v[b, row, csl] = rows_v[b, row, csl] * wsp
                return 0

            lax.fori_loop(0, ngr, group_body, 0)

        def step(ci, b):
            @pl.when(ci + 1 < cpt)
            def _():
                @pl.when(ci >= 1)
                def _():
                    drain(ssems, 1 - b)
                gather(ci + 1, 1 - b)
            drain(gsems, b)
            compute(ci, b)
            scatter(ci, b)

        gather(jnp.int32(0), 0)

        def outer(k, _):
            step(2 * k, 0)
            step(2 * k + 1, 1)
            return 0

        lax.fori_loop(0, cpt // 2, outer, 0)
        drain(ssems, 0)
        drain(ssems, 1)
        plsc.subcore_barrier()

        @pl.when(c == 0)
        def _():
            pltpu.sync_copy(acc.at[pl.ds(r0, rows_pt)], out0.at[pl.ds(r0, rows_pt)])

        @pl.when(c == 1)
        def _():
            pltpu.sync_copy(acc.at[pl.ds(r0, rows_pt)], out1.at[pl.ds(r0, rows_pt)])

    return segsum


def _row_grid(n, d, n_in, body, out_shape=None, block_rows=2000,
              out_dtype=jnp.float32):
    grid = (pl.cdiv(n, block_rows),)
    in_specs = [pl.BlockSpec((block_rows, d), lambda i: (i, 0))] * n_in
    if out_shape is None:
        out_shape = (n, d)
    out_spec = pl.BlockSpec((block_rows, out_shape[1]), lambda i: (i, 0))
    return pl.pallas_call(
        body,
        grid=grid,
        in_specs=in_specs,
        out_specs=out_spec,
        out_shape=jax.ShapeDtypeStruct(out_shape, out_dtype),
    )


def _x1_body(x0, a00, a01, o):
    o[...] = x0[...] + (a00[...] + a01[...]) * jnp.float32(1.0 / 3.0)


def _x2_body(x0, a00, a01, a10, a11, o):
    o[...] = x0[...] + (a10[...] + a11[...]) - (a00[...] + a01[...]) * jnp.float32(1.0 / 3.0)


def _g_body(x0, x1, x2, a00, a01, a10, a11, a20, a21, o):
    x3 = (x0[...] + (a00[...] + a01[...]) - (a10[...] + a11[...])
          + (a20[...] + a21[...]))
    d = x0.shape[1]
    o[:, 0 * d:1 * d] = (x0[...] * jnp.float32(_W0)).astype(o.dtype)
    o[:, 1 * d:2 * d] = (x1[...] * jnp.float32(_W1)).astype(o.dtype)
    o[:, 2 * d:3 * d] = (x2[...] * jnp.float32(_W1)).astype(o.dtype)
    o[:, 3 * d:4 * d] = (x3 * jnp.float32(_W0)).astype(o.dtype)


def _mm_body(gi, gj, o):
    o[...] = lax.dot_general(
        gi[...], gj[...], (((1,), (1,)), ((), ())),
        preferred_element_type=jnp.float32,
    )


@functools.lru_cache(maxsize=None)
def _make_gram(n, k, bm, bn):
    grid = (pl.cdiv(n, bm), pl.cdiv(n, bn))
    return pl.pallas_call(
        _mm_body,
        grid=grid,
        in_specs=[
            pl.BlockSpec((bm, k), lambda i, j: (i, 0)),
            pl.BlockSpec((bn, k), lambda i, j: (j, 0)),
        ],
        out_specs=pl.BlockSpec((bm, bn), lambda i, j: (i, j)),
        out_shape=jax.ShapeDtypeStruct((n, n), jnp.float32),
        compiler_params=pltpu.CompilerParams(
            dimension_semantics=("parallel", "parallel"),
        ),
    )


def kernel(h, e, edge_index):
    n, d = h.shape
    ev = e.shape[0]
    src = edge_index[0]
    dst = edge_index[1]
    ew = e.reshape(ev)
    np_ = ((n + NS * 8 - 1) // (NS * 8)) * (NS * 8)
    zeros = jnp.zeros((np_, d), jnp.float32)

    tch = -(-ev // _CHUNK)
    tch = -(-tch // (NW * 2)) * (NW * 2)
    evp = tch * _CHUNK
    if evp > ev:
        padn = evp - ev
        pad_dst = jnp.arange(padn, dtype=jnp.int32) % n
        pad_src = (pad_dst + 1) % n
        src = jnp.concatenate([src, pad_src])
        dst = jnp.concatenate([dst, pad_dst])
        ew = jnp.concatenate([ew, jnp.zeros((padn,), jnp.float32)])
    cpt = tch // NW
    src2 = src.reshape(NW, cpt, _CHUNK)
    dst2 = dst.reshape(NW, cpt, _CHUNK)
    ew2 = ew.reshape(NW, cpt, _CHUNK)

    segsum = _make_segsum(n, tch, d)
    a00, a01 = segsum(h, src2, dst2, ew2, zeros)
    x1 = _row_grid(n, d, 3, _x1_body)(h, a00, a01)
    a10, a11 = segsum(x1, src2, dst2, ew2, zeros)
    x2 = _row_grid(n, d, 5, _x2_body)(h, a00, a01, a10, a11)
    a20, a21 = segsum(x2, src2, dst2, ew2, zeros)
    g = _row_grid(n, d, 9, _g_body, out_shape=(n, 4 * d),
                  out_dtype=jnp.bfloat16)(
        h, x1, x2, a00, a01, a10, a11, a20, a21)
    return _make_gram(n, 4 * d, 2048, 2048)(g, g)


# speedup vs baseline: 2.0887x; 1.0352x over previous
import functools

import jax
import jax.numpy as jnp
from jax import lax
from jax.experimental import pallas as pl
from jax.experimental.pallas import tpu as pltpu
from jax.experimental.pallas import tpu_sc as plsc

NC = 2
NS = 16
NW = NC * NS

_W0 = 0.3535533905932738
_W1 = 0.6123724356957945

_CHUNK = 128
_HALF = _CHUNK // 2


@functools.lru_cache(maxsize=None)
def _make_segsum(n, tch, d):
    assert tch % NW == 0
    cpt = tch // NW
    assert cpt % 2 == 0
    np_ = ((n + NS * 8 - 1) // (NS * 8)) * (NS * 8)
    rows_pt = np_ // NS
    assert d % 16 == 0
    ngr = _CHUNK // 16

    mesh = plsc.VectorSubcoreMesh(core_axis_name="c", subcore_axis_name="s")

    @functools.partial(
        pl.kernel,
        out_type=[
            jax.ShapeDtypeStruct((np_, d), jnp.float32),
            jax.ShapeDtypeStruct((np_, d), jnp.float32),
        ],
        mesh=mesh,
        scratch_types=[
            pltpu.VMEM_SHARED((np_, d), jnp.float32),
            pltpu.VMEM((cpt, _CHUNK), jnp.int32),
            pltpu.VMEM((cpt, _CHUNK), jnp.int32),
            pltpu.VMEM((cpt, _CHUNK), jnp.float32),
            pltpu.VMEM((2, _CHUNK, d), jnp.float32),
            pltpu.SemaphoreType.DMA((2,)),
            pltpu.SemaphoreType.DMA((2,)),
        ],
    )
    def segsum(x_hbm, src_hbm, dst_hbm, e_hbm, z_hbm,
               out0, out1, acc, src_v, dst_v, w_v, rows_v, gsems, ssems):
        c = lax.axis_index("c")
        s = lax.axis_index("s")
        wid = s * NC + c
        r0 = s * rows_pt

        def gather(ci, b):
            pltpu.async_copy(x_hbm.at[src_v.at[ci]], rows_v.at[b], gsems.at[b])

        def scatter(ci, b):
            pltpu.async_copy(rows_v.at[b], acc.at[dst_v.at[ci]],
                             ssems.at[b], add=True)

        def drain(semref, b):
            pltpu.make_async_copy(x_hbm.at[pl.ds(0, _CHUNK)],
                                  rows_v.at[b], semref.at[b]).wait()

        zcp = pltpu.async_copy(z_hbm.at[pl.ds(r0, rows_pt)],
                               acc.at[pl.ds(r0, rows_pt)], ssems.at[0])
        scp = pltpu.async_copy(src_hbm.at[wid], src_v, gsems.at[0])
        dcp = pltpu.async_copy(dst_hbm.at[wid], dst_v, gsems.at[1])
        ecp = pltpu.async_copy(e_hbm.at[wid], w_v, ssems.at[1])
        scp.wait()
        gather(jnp.int32(0), 0)
        dcp.wait()
        ecp.wait()

        def wbody(ci, _):
            for g in range(ngr):
                sl = pl.ds(g * 16, 16)
                w_v[ci, sl] = jnp.where(src_v[ci, sl] == dst_v[ci, sl],
                                        jnp.float32(-1.0), w_v[ci, sl])
            return 0

        lax.fori_loop(0, cpt, wbody, 0)
        zcp.wait()
        plsc.subcore_barrier()

        def compute(ci, b):
            def group_body(g, _):
                sl = pl.ds(g * 16, 16)
                wv = w_v[ci, sl]
                for j in range(16):
                    wsp = jnp.full((16,), wv[j], jnp.float32)
                    row = g * 16 + j
                    for q in range(d // 16):
                        csl = pl.ds(q * 16, 16)
                        rows_v[b, row, csl] = rows_v[b, row, csl] * wsp
                return 0

            lax.fori_loop(0, ngr, group_body, 0)

        def step(ci, b):
            @pl.when(ci + 1 < cpt)
            def _():
                @pl.when(ci >= 1)
                def _():
                    drain(ssems, 1 - b)
                gather(ci + 1, 1 - b)
            drain(gsems, b)
            compute(ci, b)
            scatter(ci, b)

        def outer(k, _):
            step(2 * k, 0)
            step(2 * k + 1, 1)
            return 0

        lax.fori_loop(0, cpt // 2, outer, 0)
        drain(ssems, 0)
        drain(ssems, 1)
        plsc.subcore_barrier()

        @pl.when(c == 0)
        def _():
            pltpu.sync_copy(acc.at[pl.ds(r0, rows_pt)], out0.at[pl.ds(r0, rows_pt)])

        @pl.when(c == 1)
        def _():
            pltpu.sync_copy(acc.at[pl.ds(r0, rows_pt)], out1.at[pl.ds(r0, rows_pt)])

    return segsum


def _row_grid(n, d, n_in, body, out_shape=None, block_rows=2000,
              out_dtype=jnp.float32):
    grid = (pl.cdiv(n, block_rows),)
    in_specs = [pl.BlockSpec((block_rows, d), lambda i: (i, 0))] * n_in
    if out_shape is None:
        out_shape = (n, d)
    out_spec = pl.BlockSpec((block_rows, out_shape[1]), lambda i: (i, 0))
    return pl.pallas_call(
        body,
        grid=grid,
        in_specs=in_specs,
        out_specs=out_spec,
        out_shape=jax.ShapeDtypeStruct(out_shape, out_dtype),
    )


def _x1_body(x0, a00, a01, o):
    o[...] = x0[...] + (a00[...] + a01[...]) * jnp.float32(1.0 / 3.0)


def _x2_body(x0, a00, a01, a10, a11, o):
    o[...] = x0[...] + (a10[...] + a11[...]) - (a00[...] + a01[...]) * jnp.float32(1.0 / 3.0)


def _g_body(x0, x1, x2, a00, a01, a10, a11, a20, a21, o):
    x3 = (x0[...] + (a00[...] + a01[...]) - (a10[...] + a11[...])
          + (a20[...] + a21[...]))
    d = x0.shape[1]
    o[:, 0 * d:1 * d] = (x0[...] * jnp.float32(_W0)).astype(o.dtype)
    o[:, 1 * d:2 * d] = (x1[...] * jnp.float32(_W1)).astype(o.dtype)
    o[:, 2 * d:3 * d] = (x2[...] * jnp.float32(_W1)).astype(o.dtype)
    o[:, 3 * d:4 * d] = (x3 * jnp.float32(_W0)).astype(o.dtype)


def _mm_body(gi, gj, o):
    o[...] = lax.dot_general(
        gi[...], gj[...], (((1,), (1,)), ((), ())),
        preferred_element_type=jnp.float32,
    )


@functools.lru_cache(maxsize=None)
def _make_gram(n, k, bm, bn):
    grid = (pl.cdiv(n, bm), pl.cdiv(n, bn))
    return pl.pallas_call(
        _mm_body,
        grid=grid,
        in_specs=[
            pl.BlockSpec((bm, k), lambda i, j: (i, 0)),
            pl.BlockSpec((bn, k), lambda i, j: (j, 0)),
        ],
        out_specs=pl.BlockSpec((bm, bn), lambda i, j: (i, j)),
        out_shape=jax.ShapeDtypeStruct((n, n), jnp.float32),
        compiler_params=pltpu.CompilerParams(
            dimension_semantics=("parallel", "parallel"),
        ),
    )


def kernel(h, e, edge_index):
    n, d = h.shape
    ev = e.shape[0]
    src = edge_index[0]
    dst = edge_index[1]
    ew = e.reshape(ev)
    np_ = ((n + NS * 8 - 1) // (NS * 8)) * (NS * 8)
    zeros = jnp.zeros((np_, d), jnp.float32)

    tch = -(-ev // _CHUNK)
    tch = -(-tch // (NW * 2)) * (NW * 2)
    evp = tch * _CHUNK
    if evp > ev:
        padn = evp - ev
        pad_dst = jnp.arange(padn, dtype=jnp.int32) % n
        pad_src = (pad_dst + 1) % n
        src = jnp.concatenate([src, pad_src])
        dst = jnp.concatenate([dst, pad_dst])
        ew = jnp.concatenate([ew, jnp.zeros((padn,), jnp.float32)])
    cpt = tch // NW
    src2 = src.reshape(NW, cpt, _CHUNK)
    dst2 = dst.reshape(NW, cpt, _CHUNK)
    ew2 = ew.reshape(NW, cpt, _CHUNK)

    segsum = _make_segsum(n, tch, d)
    a00, a01 = segsum(h, src2, dst2, ew2, zeros)
    x1 = _row_grid(n, d, 3, _x1_body)(h, a00, a01)
    a10, a11 = segsum(x1, src2, dst2, ew2, zeros)
    x2 = _row_grid(n, d, 5, _x2_body)(h, a00, a01, a10, a11)
    a20, a21 = segsum(x2, src2, dst2, ew2, zeros)
    g = _row_grid(n, d, 9, _g_body, out_shape=(n, 4 * d),
                  out_dtype=jnp.bfloat16)(
        h, x1, x2, a00, a01, a10, a11, a20, a21)
    return _make_gram(n, 4 * d, 2048, 2048)(g, g)


# speedup vs baseline: 2.0898x; 1.0006x over previous
import functools

import jax
import jax.numpy as jnp
from jax import lax
from jax.experimental import pallas as pl
from jax.experimental.pallas import tpu as pltpu
from jax.experimental.pallas import tpu_sc as plsc

NC = 2
NS = 16
NW = NC * NS

_W0 = 0.3535533905932738
_W1 = 0.6123724356957945

_CHUNK = 128
_HALF = _CHUNK // 2


@functools.lru_cache(maxsize=None)
def _make_segsum(n, tch, d):
    assert tch % NW == 0
    cpt = tch // NW
    assert cpt % 2 == 0
    np_ = ((n + NS * 8 - 1) // (NS * 8)) * (NS * 8)
    rows_pt = np_ // NS
    assert d % 16 == 0
    ngr = _CHUNK // 16

    mesh = plsc.VectorSubcoreMesh(core_axis_name="c", subcore_axis_name="s")

    @functools.partial(
        pl.kernel,
        out_type=[
            jax.ShapeDtypeStruct((np_, d), jnp.float32),
            jax.ShapeDtypeStruct((np_, d), jnp.float32),
        ],
        mesh=mesh,
        scratch_types=[
            pltpu.VMEM_SHARED((np_, d), jnp.float32),
            pltpu.VMEM((cpt, _CHUNK), jnp.int32),
            pltpu.VMEM((cpt, _CHUNK), jnp.int32),
            pltpu.VMEM((cpt, _CHUNK), jnp.float32),
            pltpu.VMEM((2, _CHUNK, d), jnp.float32),
            pltpu.SemaphoreType.DMA((2,)),
            pltpu.SemaphoreType.DMA((2,)),
        ],
    )
    def segsum(x_hbm, src_hbm, dst_hbm, e_hbm, z_hbm,
               out0, out1, acc, src_v, dst_v, w_v, rows_v, gsems, ssems):
        c = lax.axis_index("c")
        s = lax.axis_index("s")
        wid = s * NC + c
        r0 = s * rows_pt

        def gather(ci, b):
            pltpu.async_copy(x_hbm.at[src_v.at[ci]], rows_v.at[b], gsems.at[b])

        def scatter(ci, b):
            pltpu.async_copy(rows_v.at[b], acc.at[dst_v.at[ci]],
                             ssems.at[b], add=True)

        def drain(semref, b):
            pltpu.make_async_copy(x_hbm.at[pl.ds(0, _CHUNK)],
                                  rows_v.at[b], semref.at[b]).wait()

        zcp = pltpu.async_copy(z_hbm.at[pl.ds(r0, rows_pt)],
                               acc.at[pl.ds(r0, rows_pt)], ssems.at[0])
        scp = pltpu.async_copy(src_hbm.at[wid], src_v, gsems.at[0])
        dcp = pltpu.async_copy(dst_hbm.at[wid], dst_v, gsems.at[1])
        ecp = pltpu.async_copy(e_hbm.at[wid], w_v, ssems.at[1])
        scp.wait()
        gather(jnp.int32(0), 0)
        dcp.wait()
        ecp.wait()

        def wbody(ci, _):
            for g in range(ngr):
                sl = pl.ds(g * 16, 16)
                w_v[ci, sl] = jnp.where(src_v[ci, sl] == dst_v[ci, sl],
                                        jnp.float32(-1.0), w_v[ci, sl])
            return 0

        lax.fori_loop(0, cpt, wbody, 0)
        zcp.wait()
        plsc.subcore_barrier()

        def compute(ci, b):
            def group_body(g, _):
                sl = pl.ds(g * 16, 16)
                wv = w_v[ci, sl]
                for j in range(16):
                    wsp = jnp.full((16,), wv[j], jnp.float32)
                    row = g * 16 + j
                    for q in range(d // 16):
                        csl = pl.ds(q * 16, 16)
                        rows_v[b, row, csl] = rows_v[b, row, csl] * wsp
                return 0

            lax.fori_loop(0, ngr, group_body, 0)

        def step(ci, b):
            @pl.when(ci + 1 < cpt)
            def _():
                @pl.when(ci >= 1)
                def _():
                    drain(ssems, 1 - b)
                gather(ci + 1, 1 - b)
            drain(gsems, b)
            compute(ci, b)
            scatter(ci, b)

        def outer(k, _):
            step(2 * k, 0)
            step(2 * k + 1, 1)
            return 0

        lax.fori_loop(0, cpt // 2, outer, 0)
        drain(ssems, 0)
        drain(ssems, 1)
        plsc.subcore_barrier()

        @pl.when(c == 0)
        def _():
            pltpu.sync_copy(acc.at[pl.ds(r0, rows_pt)], out0.at[pl.ds(r0, rows_pt)])

        @pl.when(c == 1)
        def _():
            pltpu.sync_copy(acc.at[pl.ds(r0, rows_pt)], out1.at[pl.ds(r0, rows_pt)])

    return segsum


def _row_grid(n, d, n_in, body, out_shape=None, block_rows=2000,
              out_dtype=jnp.float32):
    grid = (pl.cdiv(n, block_rows),)
    in_specs = [pl.BlockSpec((block_rows, d), lambda i: (i, 0))] * n_in
    if out_shape is None:
        out_shape = (n, d)
    out_spec = pl.BlockSpec((block_rows, out_shape[1]), lambda i: (i, 0))
    return pl.pallas_call(
        body,
        grid=grid,
        in_specs=in_specs,
        out_specs=out_spec,
        out_shape=jax.ShapeDtypeStruct(out_shape, out_dtype),
    )


def _x1_body(x0, a00, a01, o):
    o[...] = x0[...] + (a00[...] + a01[...]) * jnp.float32(1.0 / 3.0)


def _x2_body(x0, a00, a01, a10, a11, o):
    o[...] = x0[...] + (a10[...] + a11[...]) - (a00[...] + a01[...]) * jnp.float32(1.0 / 3.0)


def _g_body(x0, x1, x2, a00, a01, a10, a11, a20, a21, o):
    x3 = (x0[...] + (a00[...] + a01[...]) - (a10[...] + a11[...])
          + (a20[...] + a21[...]))
    d = x0.shape[1]
    o[:, 0 * d:1 * d] = (x0[...] * jnp.float32(_W0)).astype(o.dtype)
    o[:, 1 * d:2 * d] = (x1[...] * jnp.float32(_W1)).astype(o.dtype)
    o[:, 2 * d:3 * d] = (x2[...] * jnp.float32(_W1)).astype(o.dtype)
    o[:, 3 * d:4 * d] = (x3 * jnp.float32(_W0)).astype(o.dtype)


def _mm_body(gi, gj, o):
    o[...] = lax.dot_general(
        gi[...], gj[...], (((1,), (1,)), ((), ())),
        preferred_element_type=jnp.float32,
    )


@functools.lru_cache(maxsize=None)
def _make_gram(n, k, bm, bn):
    grid = (pl.cdiv(n, bm), pl.cdiv(n, bn))
    return pl.pallas_call(
        _mm_body,
        grid=grid,
        in_specs=[
            pl.BlockSpec((bm, k), lambda i, j: (i, 0)),
            pl.BlockSpec((bn, k), lambda i, j: (j, 0)),
        ],
        out_specs=pl.BlockSpec((bm, bn), lambda i, j: (i, j)),
        out_shape=jax.ShapeDtypeStruct((n, n), jnp.float32),
        compiler_params=pltpu.CompilerParams(
            dimension_semantics=("parallel", "parallel"),
        ),
    )


def kernel(h, e, edge_index):
    n, d = h.shape
    ev = e.shape[0]
    src = edge_index[0]
    dst = edge_index[1]
    ew = e.reshape(ev)
    np_ = ((n + NS * 8 - 1) // (NS * 8)) * (NS * 8)
    zeros = jnp.zeros((np_, d), jnp.float32)

    tch = -(-ev // _CHUNK)
    tch = -(-tch // (NW * 2)) * (NW * 2)
    evp = tch * _CHUNK
    if evp > ev:
        padn = evp - ev
        pad_dst = jnp.arange(padn, dtype=jnp.int32) % n
        pad_src = (pad_dst + 1) % n
        src = jnp.concatenate([src, pad_src])
        dst = jnp.concatenate([dst, pad_dst])
        ew = jnp.concatenate([ew, jnp.zeros((padn,), jnp.float32)])
    cpt = tch // NW
    src2 = src.reshape(NW, cpt, _CHUNK)
    dst2 = dst.reshape(NW, cpt, _CHUNK)
    ew2 = ew.reshape(NW, cpt, _CHUNK)

    segsum = _make_segsum(n, tch, d)
    a00, a01 = segsum(h, src2, dst2, ew2, zeros)
    x1 = _row_grid(n, d, 3, _x1_body)(h, a00, a01)
    a10, a11 = segsum(x1, src2, dst2, ew2, zeros)
    x2 = _row_grid(n, d, 5, _x2_body)(h, a00, a01, a10, a11)
    a20, a21 = segsum(x2, src2, dst2, ew2, zeros)
    g = _row_grid(n, d, 9, _g_body, out_shape=(n, 4 * d),
                  out_dtype=jnp.bfloat16)(
        h, x1, x2, a00, a01, a10, a11, a20, a21)
    return _make_gram(n, 4 * d, 2048, 2560)(g, g)
